# Initial kernel scaffold; baseline (speedup 1.0000x reference)
import jax
import jax.numpy as jnp
from jax.experimental import pallas as pl


def kernel(points_xyz, features, W0, b0, g0, be0, W1, b1, g1, be1):
    raise NotImplementedError("write your pallas kernel here")


# speedup vs baseline: 25.3674x; 25.3674x over previous
import functools

import jax
import jax.numpy as jnp
import numpy as np
from jax import lax
from jax.experimental import pallas as pl
from jax.experimental.pallas import tpu as pltpu
from jax.experimental.pallas import tpu_sc as plsc

B, N, C = 8, 8192, 64
M = 1024
S = 32
R2 = np.float32(0.2 ** 2)
EPS = 1e-5

MB = 256
NC_CHUNK = 512
N_CHUNKS = N // NC_CHUNK
C_H, C_OUT = 64, 128

SC_NC, SC_NS = 2, 16
NW = SC_NC * SC_NS
ROWS_TOTAL = B * S * M
ROWS_PER_W = ROWS_TOTAL // NW
GCHUNK = 128
GCHUNKS = ROWS_PER_W // GCHUNK


def _fps_body(xs_ref, ys_ref, zs_ref, idx_ref, nx_ref, ny_ref, nz_ref):
    xs = xs_ref[...]
    ys = ys_ref[...]
    zs = zs_ref[...]
    io_n = lax.broadcasted_iota(jnp.int32, (B, N), 1).astype(jnp.float32)
    io_m = lax.broadcasted_iota(jnp.int32, (B, M), 1)

    def body(i, state):
        dists, far, idx, nx, ny, nz = state
        eq = io_n == far
        cx = jnp.sum(jnp.where(eq, xs, 0.0), axis=1, keepdims=True)
        cy = jnp.sum(jnp.where(eq, ys, 0.0), axis=1, keepdims=True)
        cz = jnp.sum(jnp.where(eq, zs, 0.0), axis=1, keepdims=True)
        col = io_m == i
        idx = jnp.where(col, far, idx)
        nx = jnp.where(col, cx, nx)
        ny = jnp.where(col, cy, ny)
        nz = jnp.where(col, cz, nz)
        dx = xs - cx
        dy = ys - cy
        dz = zs - cz
        d = (dx * dx + dy * dy) + dz * dz
        dists = jnp.minimum(dists, d)
        mx = jnp.max(dists, axis=1, keepdims=True)
        far = jnp.min(jnp.where(dists == mx, io_n, float(N)), axis=1,
                      keepdims=True)
        return dists, far, idx, nx, ny, nz

    io_bn = lax.broadcasted_iota(jnp.int32, (B, N), 0).astype(jnp.float32)
    io_bm = lax.broadcasted_iota(jnp.int32, (B, M), 0).astype(jnp.float32)
    dists0 = io_bn * 0.0 + 1e10
    far0 = jnp.min(io_bn, axis=1, keepdims=True) * 0.0
    idx0 = io_bm * 0.0
    _, _, idx, nx, ny, nz = lax.fori_loop(
        0, M, body, (dists0, far0, idx0, idx0, idx0, idx0))
    idx_ref[...] = idx.astype(jnp.int32)
    nx_ref[...] = nx
    ny_ref[...] = ny
    nz_ref[...] = nz


def _fps(xs, ys, zs):
    return pl.pallas_call(
        _fps_body,
        out_shape=(
            jax.ShapeDtypeStruct((B, M), jnp.int32),
            jax.ShapeDtypeStruct((B, M), jnp.float32),
            jax.ShapeDtypeStruct((B, M), jnp.float32),
            jax.ShapeDtypeStruct((B, M), jnp.float32),
        ),
    )(xs, ys, zs)


def _ballq_body(xc_ref, yc_ref, zc_ref, cx_ref, cy_ref, cz_ref, out_ref):
    b = pl.program_id(0)
    cx = cx_ref[0]
    cy = cy_ref[0]
    cz = cz_ref[0]
    io_r = lax.broadcasted_iota(jnp.int32, (NC_CHUNK, NC_CHUNK), 0)
    io_c = lax.broadcasted_iota(jnp.int32, (NC_CHUNK, NC_CHUNK), 1)
    ltri = jnp.where(io_c <= io_r, 1.0, 0.0).astype(jnp.float32)
    io_sub = lax.broadcasted_iota(jnp.int32, (NC_CHUNK, MB), 0).astype(
        jnp.float32)
    kio = lax.broadcasted_iota(jnp.int32, (S, MB), 0).astype(jnp.float32)

    def cond(state):
        j, carry, _ = state
        return jnp.logical_and(j < N_CHUNKS, jnp.min(carry) < float(S))

    def body(state):
        j, carry, rows = state
        off = pl.multiple_of(j * NC_CHUNK, NC_CHUNK)
        xp = xc_ref[0, pl.ds(off, NC_CHUNK), :]
        yp = yc_ref[0, pl.ds(off, NC_CHUNK), :]
        zp = zc_ref[0, pl.ds(off, NC_CHUNK), :]
        dx = cx - xp
        dy = cy - yp
        dz = cz - zp
        sq = (dx * dx + dy * dy) + dz * dz
        m01 = jnp.where(sq <= R2, 1.0, 0.0).astype(jnp.float32)
        cum = jnp.dot(ltri, m01, preferred_element_type=jnp.float32) + carry
        slot = jnp.where((m01 > 0.0) & (cum <= float(S)), cum, 0.0)
        gidx = io_sub + (j * NC_CHUNK).astype(jnp.float32)
        ks = []
        for k in range(S):
            ks.append(jnp.sum(jnp.where(slot == float(k + 1), gidx, 0.0),
                              axis=0, keepdims=True))
        rows = rows + jnp.concatenate(ks, axis=0)
        carry = cum[NC_CHUNK - 1:NC_CHUNK, :]
        return j + 1, carry, rows

    j0 = jnp.int32(0)
    carry0 = jnp.zeros((1, MB), dtype=jnp.float32)
    rows0 = jnp.zeros((S, MB), dtype=jnp.float32)
    _, carry, rows = lax.while_loop(cond, body, (j0, carry0, rows0))
    cnt = jnp.minimum(carry, float(S))
    first = rows[0:1, :]
    rows = jnp.where(kio < cnt, rows, first)
    out_ref[0] = rows.astype(jnp.int32) + b * N


def _ballq(xc, yc, zc, nx, ny, nz):
    return pl.pallas_call(
        _ballq_body,
        grid=(B, M // MB),
        in_specs=[
            pl.BlockSpec((1, N, 1), lambda b, mb: (b, 0, 0)),
            pl.BlockSpec((1, N, 1), lambda b, mb: (b, 0, 0)),
            pl.BlockSpec((1, N, 1), lambda b, mb: (b, 0, 0)),
            pl.BlockSpec((1, 1, MB), lambda b, mb: (b, 0, mb)),
            pl.BlockSpec((1, 1, MB), lambda b, mb: (b, 0, mb)),
            pl.BlockSpec((1, 1, MB), lambda b, mb: (b, 0, mb)),
        ],
        out_specs=pl.BlockSpec((1, S, MB), lambda b, mb: (b, 0, mb)),
        out_shape=jax.ShapeDtypeStruct((B, S, M), jnp.int32),
    )(xc, yc, zc, nx.reshape(B, 1, M), ny.reshape(B, 1, M),
      nz.reshape(B, 1, M))


def _tmat_body(xf_ref, w_ref, out_ref):
    out_ref[0] = jnp.dot(xf_ref[0], w_ref[...],
                         preferred_element_type=jnp.float32)


def _tmat(xyzfeat, w0pt):
    return pl.pallas_call(
        _tmat_body,
        grid=(B,),
        in_specs=[
            pl.BlockSpec((1, N, 3 + C), lambda b: (b, 0, 0)),
            pl.BlockSpec((3 + C, C_H), lambda b: (0, 0)),
        ],
        out_specs=pl.BlockSpec((1, N, C_H), lambda b: (b, 0, 0)),
        out_shape=jax.ShapeDtypeStruct((B, N, C_H), jnp.float32),
    )(xyzfeat, w0pt)


def _sc_gather(table, idx3):
    mesh = plsc.VectorSubcoreMesh(core_axis_name="c", subcore_axis_name="s")

    @functools.partial(
        pl.kernel,
        mesh=mesh,
        compiler_params=pltpu.CompilerParams(use_tc_tiling_on_sc=False),
        out_type=jax.ShapeDtypeStruct((ROWS_TOTAL, C_H), jnp.float32),
        scratch_types=[
            pltpu.VMEM((GCHUNKS, GCHUNK), jnp.int32),
            pltpu.VMEM((GCHUNK, C_H), jnp.float32),
            pltpu.SemaphoreType.DMA,
        ],
    )
    def k(table_hbm, idx_hbm, out_hbm, idx_v, buf, sem):
        wid = lax.axis_index("s") * SC_NC + lax.axis_index("c")
        pltpu.sync_copy(idx_hbm.at[wid], idx_v)
        base = wid * ROWS_PER_W

        def body(j, carry):
            pltpu.async_copy(table_hbm.at[idx_v.at[j]], buf, sem).wait()
            pltpu.sync_copy(buf, out_hbm.at[pl.ds(base + j * GCHUNK, GCHUNK)])
            return carry

        lax.fori_loop(0, GCHUNKS, body, 0)

    return k(table, idx3)


def _mlp_body(g_ref, nxyz_ref, w1t_ref, bias1_ref, t0_ref, wx_ref, wy_ref,
              wz_ref, out_ref):
    nb = nxyz_ref[0]
    cx = nb[:, 0:1]
    cy = nb[:, 1:2]
    cz = nb[:, 2:3]
    c = t0_ref[...] - (cx * wx_ref[...] + cy * wy_ref[...] + cz * wz_ref[...])
    g = g_ref[0]
    h = jnp.maximum(g + c[None, :, :], 0.0)
    h2 = h.reshape(S * MB, C_H)
    z = jnp.dot(h2, w1t_ref[...], preferred_element_type=jnp.float32)
    z = jnp.maximum(z + bias1_ref[...], 0.0)
    out_ref[0] = jnp.max(z.reshape(S, MB, C_OUT), axis=0)


def _mlp(gathered, new_xyz, w1t, bias1, t0, wx, wy, wz):
    return pl.pallas_call(
        _mlp_body,
        grid=(B, M // MB),
        in_specs=[
            pl.BlockSpec((1, S, MB, C_H), lambda b, mb: (b, 0, mb, 0)),
            pl.BlockSpec((1, MB, 3), lambda b, mb: (b, mb, 0)),
            pl.BlockSpec((C_H, C_OUT), lambda b, mb: (0, 0)),
            pl.BlockSpec((1, C_OUT), lambda b, mb: (0, 0)),
            pl.BlockSpec((1, C_H), lambda b, mb: (0, 0)),
            pl.BlockSpec((1, C_H), lambda b, mb: (0, 0)),
            pl.BlockSpec((1, C_H), lambda b, mb: (0, 0)),
            pl.BlockSpec((1, C_H), lambda b, mb: (0, 0)),
        ],
        out_specs=pl.BlockSpec((1, MB, C_OUT), lambda b, mb: (b, mb, 0)),
        out_shape=jax.ShapeDtypeStruct((B, M, C_OUT), jnp.float32),
    )(gathered, new_xyz, w1t, bias1, t0, wx, wy, wz)


def kernel(points_xyz, features, W0, b0, g0, be0, W1, b1, g1, be1):
    xs = points_xyz[:, :, 0]
    ys = points_xyz[:, :, 1]
    zs = points_xyz[:, :, 2]

    fps_idx, nx, ny, nz = _fps(xs, ys, zs)
    new_xyz = jnp.stack([nx, ny, nz], axis=-1)

    xc = points_xyz[:, :, 0:1]
    yc = points_xyz[:, :, 1:2]
    zc = points_xyz[:, :, 2:3]
    gidx = _ballq(xc, yc, zc, nx, ny, nz)

    inv = 1.0 / jnp.sqrt(jnp.float32(1.0 + EPS))
    s0 = g0 * inv
    s1 = g1 * inv
    w0p = W0 * s0[:, None]
    feats_t = jnp.transpose(features, (0, 2, 1))
    xyzfeat = jnp.concatenate([points_xyz, feats_t], axis=-1)
    tmat = _tmat(xyzfeat, w0p.T)

    table = tmat.reshape(B * N, C_H)
    idx3 = gidx.reshape(NW, GCHUNKS, GCHUNK)
    gathered = _sc_gather(table, idx3).reshape(B, S, M, C_H)

    t0 = (b0 * s0 + be0)[None, :]
    wx = w0p[:, 0][None, :]
    wy = w0p[:, 1][None, :]
    wz = w0p[:, 2][None, :]
    w1p = W1 * s1[:, None]
    bias1 = (b1 * s1 + be1)[None, :]
    feats_out = _mlp(gathered, new_xyz, w1p.T, bias1, t0, wx, wy, wz)
    new_features = jnp.transpose(feats_out, (0, 2, 1))
    return new_xyz, new_features, fps_idx


# speedup vs baseline: 26.4635x; 1.0432x over previous
import functools

import jax
import jax.numpy as jnp
import numpy as np
from jax import lax
from jax.experimental import pallas as pl
from jax.experimental.pallas import tpu as pltpu
from jax.experimental.pallas import tpu_sc as plsc

B, N, C = 8, 8192, 64
M = 1024
S = 32
R2 = np.float32(0.2 ** 2)
EPS = 1e-5

MB = 256
NC_CHUNK = 512
N_CHUNKS = N // NC_CHUNK
C_H, C_OUT = 64, 128

SC_NC, SC_NS = 2, 16
NW = SC_NC * SC_NS
ROWS_TOTAL = B * S * M
ROWS_PER_W = ROWS_TOTAL // NW
GCHUNK = 128
GCHUNKS = ROWS_PER_W // GCHUNK
GDEPTH = 8


def _fps_body(xs_ref, ys_ref, zs_ref, idx_ref, nx_ref, ny_ref, nz_ref):
    xs = xs_ref[...]
    ys = ys_ref[...]
    zs = zs_ref[...]
    io_n = lax.broadcasted_iota(jnp.int32, (B, N), 1).astype(jnp.float32)
    io_m = lax.broadcasted_iota(jnp.int32, (B, M), 1)

    def body(i, state):
        dists, far, idx, nx, ny, nz = state
        eq = io_n == far
        cx = jnp.sum(jnp.where(eq, xs, 0.0), axis=1, keepdims=True)
        cy = jnp.sum(jnp.where(eq, ys, 0.0), axis=1, keepdims=True)
        cz = jnp.sum(jnp.where(eq, zs, 0.0), axis=1, keepdims=True)
        col = io_m == i
        idx = jnp.where(col, far, idx)
        nx = jnp.where(col, cx, nx)
        ny = jnp.where(col, cy, ny)
        nz = jnp.where(col, cz, nz)
        dx = xs - cx
        dy = ys - cy
        dz = zs - cz
        d = (dx * dx + dy * dy) + dz * dz
        dists = jnp.minimum(dists, d)
        mx = jnp.max(dists, axis=1, keepdims=True)
        far = jnp.min(jnp.where(dists == mx, io_n, float(N)), axis=1,
                      keepdims=True)
        return dists, far, idx, nx, ny, nz

    io_bn = lax.broadcasted_iota(jnp.int32, (B, N), 0).astype(jnp.float32)
    io_bm = lax.broadcasted_iota(jnp.int32, (B, M), 0).astype(jnp.float32)
    dists0 = io_bn * 0.0 + 1e10
    far0 = jnp.min(io_bn, axis=1, keepdims=True) * 0.0
    idx0 = io_bm * 0.0
    _, _, idx, nx, ny, nz = lax.fori_loop(
        0, M, body, (dists0, far0, idx0, idx0, idx0, idx0))
    idx_ref[...] = idx.astype(jnp.int32)
    nx_ref[...] = nx
    ny_ref[...] = ny
    nz_ref[...] = nz


def _fps(xs, ys, zs):
    return pl.pallas_call(
        _fps_body,
        out_shape=(
            jax.ShapeDtypeStruct((B, M), jnp.int32),
            jax.ShapeDtypeStruct((B, M), jnp.float32),
            jax.ShapeDtypeStruct((B, M), jnp.float32),
            jax.ShapeDtypeStruct((B, M), jnp.float32),
        ),
    )(xs, ys, zs)


def _ballq_body(xc_ref, yc_ref, zc_ref, cx_ref, cy_ref, cz_ref, out_ref,
                rows_ref):
    b = pl.program_id(0)
    cx = cx_ref[0]
    cy = cy_ref[0]
    cz = cz_ref[0]
    io_r = lax.broadcasted_iota(jnp.int32, (NC_CHUNK, NC_CHUNK), 0)
    io_c = lax.broadcasted_iota(jnp.int32, (NC_CHUNK, NC_CHUNK), 1)
    ltri = jnp.where(io_c <= io_r, 1.0, 0.0).astype(jnp.float32)
    io_sub = lax.broadcasted_iota(jnp.int32, (NC_CHUNK, MB), 0).astype(
        jnp.float32)
    kio = lax.broadcasted_iota(jnp.int32, (S, MB), 0).astype(jnp.float32)

    rows_ref[...] = kio * 0.0

    def cond(state):
        j, carry = state
        return jnp.logical_and(j < N_CHUNKS, jnp.min(carry) < float(S))

    def body(state):
        j, carry = state
        off = pl.multiple_of(j * NC_CHUNK, NC_CHUNK)
        xp = xc_ref[0, pl.ds(off, NC_CHUNK), :]
        yp = yc_ref[0, pl.ds(off, NC_CHUNK), :]
        zp = zc_ref[0, pl.ds(off, NC_CHUNK), :]
        dx = cx - xp
        dy = cy - yp
        dz = cz - zp
        sq = (dx * dx + dy * dy) + dz * dz
        m01 = jnp.where(sq <= R2, 1.0, 0.0).astype(jnp.float32)
        cum = jnp.dot(ltri, m01, preferred_element_type=jnp.float32) + carry
        slot = jnp.where((m01 > 0.0) & (cum <= float(S)), cum, 0.0)
        gidx = io_sub + (j * NC_CHUNK).astype(jnp.float32)
        for k in range(S):
            v = jnp.sum(jnp.where(slot == float(k + 1), gidx, 0.0),
                        axis=0, keepdims=True)
            rows_ref[k:k + 1, :] += v
        carry = cum[NC_CHUNK - 1:NC_CHUNK, :]
        return j + 1, carry

    j0 = jnp.int32(0)
    carry0 = jnp.zeros((1, MB), dtype=jnp.float32)
    _, carry = lax.while_loop(cond, body, (j0, carry0))
    rows = rows_ref[...]
    cnt = jnp.minimum(carry, float(S))
    first = rows[0:1, :]
    rows = jnp.where(kio < cnt, rows, first)
    out_ref[0] = rows.astype(jnp.int32) + b * N


def _ballq(xc, yc, zc, nx, ny, nz):
    return pl.pallas_call(
        _ballq_body,
        grid=(B, M // MB),
        in_specs=[
            pl.BlockSpec((1, N, 1), lambda b, mb: (b, 0, 0)),
            pl.BlockSpec((1, N, 1), lambda b, mb: (b, 0, 0)),
            pl.BlockSpec((1, N, 1), lambda b, mb: (b, 0, 0)),
            pl.BlockSpec((1, 1, MB), lambda b, mb: (b, 0, mb)),
            pl.BlockSpec((1, 1, MB), lambda b, mb: (b, 0, mb)),
            pl.BlockSpec((1, 1, MB), lambda b, mb: (b, 0, mb)),
        ],
        out_specs=pl.BlockSpec((1, S, MB), lambda b, mb: (b, 0, mb)),
        out_shape=jax.ShapeDtypeStruct((B, S, M), jnp.int32),
        scratch_shapes=[pltpu.VMEM((S, MB), jnp.float32)],
    )(xc, yc, zc, nx.reshape(B, 1, M), ny.reshape(B, 1, M),
      nz.reshape(B, 1, M))


def _tmat_body(f_ref, xyz_ref, wf_ref, wx_ref, out_ref):
    a = lax.dot_general(f_ref[0], wf_ref[...], (((0,), (0,)), ((), ())),
                        preferred_element_type=jnp.float32)
    bpart = jnp.dot(xyz_ref[0], wx_ref[...],
                    preferred_element_type=jnp.float32)
    out_ref[0] = a + bpart


def _tmat(features, points_xyz, wf, wx):
    return pl.pallas_call(
        _tmat_body,
        grid=(B,),
        in_specs=[
            pl.BlockSpec((1, C, N), lambda b: (b, 0, 0)),
            pl.BlockSpec((1, N, 3), lambda b: (b, 0, 0)),
            pl.BlockSpec((C, C_H), lambda b: (0, 0)),
            pl.BlockSpec((3, C_H), lambda b: (0, 0)),
        ],
        out_specs=pl.BlockSpec((1, N, C_H), lambda b: (b, 0, 0)),
        out_shape=jax.ShapeDtypeStruct((B, N, C_H), jnp.float32),
    )(features, points_xyz, wf, wx)


def _sc_gather(table, idx3):
    mesh = plsc.VectorSubcoreMesh(core_axis_name="c", subcore_axis_name="s")

    @functools.partial(
        pl.kernel,
        mesh=mesh,
        compiler_params=pltpu.CompilerParams(use_tc_tiling_on_sc=False),
        out_type=jax.ShapeDtypeStruct((ROWS_TOTAL, C_H), jnp.float32),
        scratch_types=[
            pltpu.VMEM((GCHUNKS, GCHUNK), jnp.int32),
            pltpu.VMEM((GDEPTH, GCHUNK, C_H), jnp.float32),
            pltpu.SemaphoreType.DMA,
        ],
    )
    def k(table_hbm, idx_hbm, out_hbm, idx_v, bufs, sem):
        wid = lax.axis_index("s") * SC_NC + lax.axis_index("c")
        pltpu.sync_copy(idx_hbm.at[wid], idx_v)
        base = wid * ROWS_PER_W

        def rnd(r, carry):
            waits = []
            for d in range(GDEPTH):
                waits.append(pltpu.async_copy(
                    table_hbm.at[idx_v.at[r * GDEPTH + d]], bufs.at[d], sem))
            for d in range(GDEPTH):
                waits[d].wait()
                j = r * GDEPTH + d
                pltpu.sync_copy(
                    bufs.at[d], out_hbm.at[pl.ds(base + j * GCHUNK, GCHUNK)])
            return carry

        lax.fori_loop(0, GCHUNKS // GDEPTH, rnd, 0)

    return k(table, idx3)


def _mlp_body(g_ref, nxyz_ref, w1t_ref, bias1_ref, t0_ref, wx_ref, wy_ref,
              wz_ref, out_ref):
    nb = nxyz_ref[0]
    cx = nb[:, 0:1]
    cy = nb[:, 1:2]
    cz = nb[:, 2:3]
    c = t0_ref[...] - (cx * wx_ref[...] + cy * wy_ref[...] + cz * wz_ref[...])
    g = g_ref[0]
    h = jnp.maximum(g + c[None, :, :], 0.0)
    h2 = h.reshape(S * MB, C_H)
    z = jnp.dot(h2, w1t_ref[...], preferred_element_type=jnp.float32)
    z = jnp.maximum(z + bias1_ref[...], 0.0)
    out_ref[0] = jnp.max(z.reshape(S, MB, C_OUT), axis=0)


def _mlp(gathered, new_xyz, w1t, bias1, t0, wx, wy, wz):
    return pl.pallas_call(
        _mlp_body,
        grid=(B, M // MB),
        in_specs=[
            pl.BlockSpec((1, S, MB, C_H), lambda b, mb: (b, 0, mb, 0)),
            pl.BlockSpec((1, MB, 3), lambda b, mb: (b, mb, 0)),
            pl.BlockSpec((C_H, C_OUT), lambda b, mb: (0, 0)),
            pl.BlockSpec((1, C_OUT), lambda b, mb: (0, 0)),
            pl.BlockSpec((1, C_H), lambda b, mb: (0, 0)),
            pl.BlockSpec((1, C_H), lambda b, mb: (0, 0)),
            pl.BlockSpec((1, C_H), lambda b, mb: (0, 0)),
            pl.BlockSpec((1, C_H), lambda b, mb: (0, 0)),
        ],
        out_specs=pl.BlockSpec((1, MB, C_OUT), lambda b, mb: (b, mb, 0)),
        out_shape=jax.ShapeDtypeStruct((B, M, C_OUT), jnp.float32),
    )(gathered, new_xyz, w1t, bias1, t0, wx, wy, wz)


def kernel(points_xyz, features, W0, b0, g0, be0, W1, b1, g1, be1):
    xs = points_xyz[:, :, 0]
    ys = points_xyz[:, :, 1]
    zs = points_xyz[:, :, 2]

    fps_idx, nx, ny, nz = _fps(xs, ys, zs)
    new_xyz = jnp.stack([nx, ny, nz], axis=-1)

    xc = points_xyz[:, :, 0:1]
    yc = points_xyz[:, :, 1:2]
    zc = points_xyz[:, :, 2:3]
    gidx = _ballq(xc, yc, zc, nx, ny, nz)

    inv = 1.0 / jnp.sqrt(jnp.float32(1.0 + EPS))
    s0 = g0 * inv
    s1 = g1 * inv
    w0p = W0 * s0[:, None]
    w0pt = w0p.T
    tmat = _tmat(features, points_xyz, w0pt[3:], w0pt[:3])

    table = tmat.reshape(B * N, C_H)
    idx3 = gidx.reshape(NW, GCHUNKS, GCHUNK)
    gathered = _sc_gather(table, idx3).reshape(B, S, M, C_H)

    t0 = (b0 * s0 + be0)[None, :]
    wx = w0p[:, 0][None, :]
    wy = w0p[:, 1][None, :]
    wz = w0p[:, 2][None, :]
    w1p = W1 * s1[:, None]
    bias1 = (b1 * s1 + be1)[None, :]
    feats_out = _mlp(gathered, new_xyz, w1p.T, bias1, t0, wx, wy, wz)
    new_features = jnp.transpose(feats_out, (0, 2, 1))
    return new_xyz, new_features, fps_idx


# speedup vs baseline: 28.5285x; 1.0780x over previous
import functools

import jax
import jax.numpy as jnp
import numpy as np
from jax import lax
from jax.experimental import pallas as pl
from jax.experimental.pallas import tpu as pltpu
from jax.experimental.pallas import tpu_sc as plsc

B, N, C = 8, 8192, 64
M = 1024
S = 32
R2 = np.float32(0.2 ** 2)
EPS = 1e-5

MB = 256
NC_CHUNK = 512
N_CHUNKS = N // NC_CHUNK
C_H, C_OUT = 64, 128

SC_NC, SC_NS = 2, 16
NW = SC_NC * SC_NS
ROWS_TOTAL = B * S * M
ROWS_PER_W = ROWS_TOTAL // NW
GCHUNK = 128
GCHUNKS = ROWS_PER_W // GCHUNK
GDEPTH = 8


NR, NL = 64, 128
MR, ML = 8, 128


def _red2(v, op):
    return op(op(v, axis=0, keepdims=True), axis=1, keepdims=True)


def _fps_body(xs_ref, ys_ref, zs_ref, idx_ref, nx_ref, ny_ref, nz_ref):
    io_n = (lax.broadcasted_iota(jnp.int32, (NR, NL), 0) * NL
            + lax.broadcasted_iota(jnp.int32, (NR, NL), 1)).astype(
                jnp.float32)
    io_m = (lax.broadcasted_iota(jnp.int32, (MR, ML), 0) * ML
            + lax.broadcasted_iota(jnp.int32, (MR, ML), 1))

    def body(i, state):
        col = io_m == i
        out = []
        for b in range(B):
            dists, far, idx, nx, ny, nz = state[b]
            sel = io_n == far
            cx = _red2(jnp.where(sel, xs_ref[b], 0.0), jnp.sum)
            cy = _red2(jnp.where(sel, ys_ref[b], 0.0), jnp.sum)
            cz = _red2(jnp.where(sel, zs_ref[b], 0.0), jnp.sum)
            idx = jnp.where(col, far, idx)
            nx = jnp.where(col, cx, nx)
            ny = jnp.where(col, cy, ny)
            nz = jnp.where(col, cz, nz)
            dx = xs_ref[b] - cx
            dy = ys_ref[b] - cy
            dz = zs_ref[b] - cz
            d = (dx * dx + dy * dy) + dz * dz
            dists = jnp.minimum(dists, d)
            mx = _red2(dists, jnp.max)
            far = _red2(jnp.where(dists == mx, io_n, float(N)), jnp.min)
            out.append((dists, far, idx, nx, ny, nz))
        return tuple(out)

    state0 = []
    for b in range(B):
        dists0 = xs_ref[b] * 0.0 + 1e10
        far0 = _red2(xs_ref[b], jnp.min) * 0.0
        acc0 = io_m.astype(jnp.float32) * 0.0
        state0.append((dists0, far0, acc0, acc0, acc0, acc0))
    state = lax.fori_loop(0, M, body, tuple(state0))
    for b in range(B):
        _, _, idx, nx, ny, nz = state[b]
        idx_ref[b] = idx.astype(jnp.int32)
        nx_ref[b] = nx
        ny_ref[b] = ny
        nz_ref[b] = nz


def _fps(xs, ys, zs):
    return pl.pallas_call(
        _fps_body,
        out_shape=(
            jax.ShapeDtypeStruct((B, MR, ML), jnp.int32),
            jax.ShapeDtypeStruct((B, MR, ML), jnp.float32),
            jax.ShapeDtypeStruct((B, MR, ML), jnp.float32),
            jax.ShapeDtypeStruct((B, MR, ML), jnp.float32),
        ),
    )(xs, ys, zs)


def _ballq_body(xc_ref, yc_ref, zc_ref, cx_ref, cy_ref, cz_ref, out_ref,
                rows_ref):
    b = pl.program_id(0)
    cx = cx_ref[0]
    cy = cy_ref[0]
    cz = cz_ref[0]
    io_r = lax.broadcasted_iota(jnp.int32, (NC_CHUNK, NC_CHUNK), 0)
    io_c = lax.broadcasted_iota(jnp.int32, (NC_CHUNK, NC_CHUNK), 1)
    ltri = jnp.where(io_c <= io_r, 1.0, 0.0).astype(jnp.float32)
    io_sub = lax.broadcasted_iota(jnp.int32, (NC_CHUNK, MB), 0).astype(
        jnp.float32)
    kio = lax.broadcasted_iota(jnp.int32, (S, MB), 0).astype(jnp.float32)

    rows_ref[...] = kio * 0.0

    def cond(state):
        j, carry = state
        return jnp.logical_and(j < N_CHUNKS, jnp.min(carry) < float(S))

    def body(state):
        j, carry = state
        off = pl.multiple_of(j * NC_CHUNK, NC_CHUNK)
        xp = xc_ref[0, pl.ds(off, NC_CHUNK), :]
        yp = yc_ref[0, pl.ds(off, NC_CHUNK), :]
        zp = zc_ref[0, pl.ds(off, NC_CHUNK), :]
        dx = cx - xp
        dy = cy - yp
        dz = cz - zp
        sq = (dx * dx + dy * dy) + dz * dz
        m01 = jnp.where(sq <= R2, 1.0, 0.0).astype(jnp.float32)
        cum = jnp.dot(ltri, m01, preferred_element_type=jnp.float32) + carry
        slot = jnp.where((m01 > 0.0) & (cum <= float(S)), cum, 0.0)
        gidx = io_sub + (j * NC_CHUNK).astype(jnp.float32)
        for k in range(S):
            v = jnp.sum(jnp.where(slot == float(k + 1), gidx, 0.0),
                        axis=0, keepdims=True)
            rows_ref[k:k + 1, :] += v
        carry = cum[NC_CHUNK - 1:NC_CHUNK, :]
        return j + 1, carry

    j0 = jnp.int32(0)
    carry0 = jnp.zeros((1, MB), dtype=jnp.float32)
    _, carry = lax.while_loop(cond, body, (j0, carry0))
    rows = rows_ref[...]
    cnt = jnp.minimum(carry, float(S))
    first = rows[0:1, :]
    rows = jnp.where(kio < cnt, rows, first)
    out_ref[0] = rows.astype(jnp.int32) + b * N


def _ballq(xc, yc, zc, nx, ny, nz):
    return pl.pallas_call(
        _ballq_body,
        grid=(B, M // MB),
        in_specs=[
            pl.BlockSpec((1, N, 1), lambda b, mb: (b, 0, 0)),
            pl.BlockSpec((1, N, 1), lambda b, mb: (b, 0, 0)),
            pl.BlockSpec((1, N, 1), lambda b, mb: (b, 0, 0)),
            pl.BlockSpec((1, 1, MB), lambda b, mb: (b, 0, mb)),
            pl.BlockSpec((1, 1, MB), lambda b, mb: (b, 0, mb)),
            pl.BlockSpec((1, 1, MB), lambda b, mb: (b, 0, mb)),
        ],
        out_specs=pl.BlockSpec((1, S, MB), lambda b, mb: (b, 0, mb)),
        out_shape=jax.ShapeDtypeStruct((B, S, M), jnp.int32),
        scratch_shapes=[pltpu.VMEM((S, MB), jnp.float32)],
    )(xc, yc, zc, nx.reshape(B, 1, M), ny.reshape(B, 1, M),
      nz.reshape(B, 1, M))


def _tmat_body(f_ref, xyz_ref, wf_ref, wx_ref, out_ref):
    a = lax.dot_general(f_ref[0], wf_ref[...], (((0,), (0,)), ((), ())),
                        preferred_element_type=jnp.float32)
    bpart = jnp.dot(xyz_ref[0], wx_ref[...],
                    preferred_element_type=jnp.float32)
    out_ref[0] = a + bpart


def _tmat(features, points_xyz, wf, wx):
    return pl.pallas_call(
        _tmat_body,
        grid=(B,),
        in_specs=[
            pl.BlockSpec((1, C, N), lambda b: (b, 0, 0)),
            pl.BlockSpec((1, N, 3), lambda b: (b, 0, 0)),
            pl.BlockSpec((C, C_H), lambda b: (0, 0)),
            pl.BlockSpec((3, C_H), lambda b: (0, 0)),
        ],
        out_specs=pl.BlockSpec((1, N, C_H), lambda b: (b, 0, 0)),
        out_shape=jax.ShapeDtypeStruct((B, N, C_H), jnp.float32),
    )(features, points_xyz, wf, wx)


def _sc_gather(table, idx3):
    mesh = plsc.VectorSubcoreMesh(core_axis_name="c", subcore_axis_name="s")

    @functools.partial(
        pl.kernel,
        mesh=mesh,
        compiler_params=pltpu.CompilerParams(use_tc_tiling_on_sc=False),
        out_type=jax.ShapeDtypeStruct((ROWS_TOTAL, C_H), jnp.float32),
        scratch_types=[
            pltpu.VMEM((GCHUNKS, GCHUNK), jnp.int32),
            pltpu.VMEM((GDEPTH, GCHUNK, C_H), jnp.float32),
            pltpu.SemaphoreType.DMA,
        ],
    )
    def k(table_hbm, idx_hbm, out_hbm, idx_v, bufs, sem):
        wid = lax.axis_index("s") * SC_NC + lax.axis_index("c")
        pltpu.sync_copy(idx_hbm.at[wid], idx_v)
        base = wid * ROWS_PER_W

        def rnd(r, carry):
            waits = []
            for d in range(GDEPTH):
                waits.append(pltpu.async_copy(
                    table_hbm.at[idx_v.at[r * GDEPTH + d]], bufs.at[d], sem))
            for d in range(GDEPTH):
                waits[d].wait()
                j = r * GDEPTH + d
                pltpu.sync_copy(
                    bufs.at[d], out_hbm.at[pl.ds(base + j * GCHUNK, GCHUNK)])
            return carry

        lax.fori_loop(0, GCHUNKS // GDEPTH, rnd, 0)

    return k(table, idx3)


def _mlp_body(g_ref, nxyz_ref, w1t_ref, bias1_ref, t0_ref, wx_ref, wy_ref,
              wz_ref, out_ref):
    nb = nxyz_ref[0]
    cx = nb[:, 0:1]
    cy = nb[:, 1:2]
    cz = nb[:, 2:3]
    c = t0_ref[...] - (cx * wx_ref[...] + cy * wy_ref[...] + cz * wz_ref[...])
    g = g_ref[0]
    h = jnp.maximum(g + c[None, :, :], 0.0)
    h2 = h.reshape(S * MB, C_H)
    z = jnp.dot(h2, w1t_ref[...], preferred_element_type=jnp.float32)
    z = jnp.maximum(z + bias1_ref[...], 0.0)
    out_ref[0] = jnp.max(z.reshape(S, MB, C_OUT), axis=0)


def _mlp(gathered, new_xyz, w1t, bias1, t0, wx, wy, wz):
    return pl.pallas_call(
        _mlp_body,
        grid=(B, M // MB),
        in_specs=[
            pl.BlockSpec((1, S, MB, C_H), lambda b, mb: (b, 0, mb, 0)),
            pl.BlockSpec((1, MB, 3), lambda b, mb: (b, mb, 0)),
            pl.BlockSpec((C_H, C_OUT), lambda b, mb: (0, 0)),
            pl.BlockSpec((1, C_OUT), lambda b, mb: (0, 0)),
            pl.BlockSpec((1, C_H), lambda b, mb: (0, 0)),
            pl.BlockSpec((1, C_H), lambda b, mb: (0, 0)),
            pl.BlockSpec((1, C_H), lambda b, mb: (0, 0)),
            pl.BlockSpec((1, C_H), lambda b, mb: (0, 0)),
        ],
        out_specs=pl.BlockSpec((1, MB, C_OUT), lambda b, mb: (b, mb, 0)),
        out_shape=jax.ShapeDtypeStruct((B, M, C_OUT), jnp.float32),
    )(gathered, new_xyz, w1t, bias1, t0, wx, wy, wz)


def kernel(points_xyz, features, W0, b0, g0, be0, W1, b1, g1, be1):
    xs = points_xyz[:, :, 0].reshape(B, NR, NL)
    ys = points_xyz[:, :, 1].reshape(B, NR, NL)
    zs = points_xyz[:, :, 2].reshape(B, NR, NL)

    fps_idx3, nx3, ny3, nz3 = _fps(xs, ys, zs)
    fps_idx = fps_idx3.reshape(B, M)
    nx = nx3.reshape(B, M)
    ny = ny3.reshape(B, M)
    nz = nz3.reshape(B, M)
    new_xyz = jnp.stack([nx, ny, nz], axis=-1)

    xc = points_xyz[:, :, 0:1]
    yc = points_xyz[:, :, 1:2]
    zc = points_xyz[:, :, 2:3]
    gidx = _ballq(xc, yc, zc, nx, ny, nz)

    inv = 1.0 / jnp.sqrt(jnp.float32(1.0 + EPS))
    s0 = g0 * inv
    s1 = g1 * inv
    w0p = W0 * s0[:, None]
    w0pt = w0p.T
    tmat = _tmat(features, points_xyz, w0pt[3:], w0pt[:3])

    table = tmat.reshape(B * N, C_H)
    idx3 = gidx.reshape(NW, GCHUNKS, GCHUNK)
    gathered = _sc_gather(table, idx3).reshape(B, S, M, C_H)

    t0 = (b0 * s0 + be0)[None, :]
    wx = w0p[:, 0][None, :]
    wy = w0p[:, 1][None, :]
    wz = w0p[:, 2][None, :]
    w1p = W1 * s1[:, None]
    bias1 = (b1 * s1 + be1)[None, :]
    feats_out = _mlp(gathered, new_xyz, w1p.T, bias1, t0, wx, wy, wz)
    new_features = jnp.transpose(feats_out, (0, 2, 1))
    return new_xyz, new_features, fps_idx
